# baseline (speedup 1.0000x reference)
import functools
import jax
import jax.numpy as jnp
from jax import lax
from jax.experimental import pallas as pl
from jax.experimental.pallas import tpu as pltpu
from jax.experimental.pallas import tpu_sc as plsc

N_NODES = 10000
N_EDGES = 320000
D = 128
ALPHA = 1.0

NPAD = 10240
NB = 1024
NBLK = NPAD // NB
NC, NS = 2, 16
NW = NC * NS
EW = N_EDGES // NW
SL = NPAD // NS
FPT = D // NW
CH = 3200
NCH = N_EDGES // CH
GRP = 10
P = 8
ESH = N_EDGES // NC
CH2 = 4000
NCH2 = ESH // CH2
NEG = -3.0e38

_mesh = plsc.VectorSubcoreMesh(core_axis_name="c", subcore_axis_name="s")


def _hist_body(src_hbm, dst_hbm, out_hbm, sbuf, dbuf, hs, hd, rbuf, od, oi, shared):
    cid = lax.axis_index("c")
    sid = lax.axis_index("s")
    wid = cid * NS + sid

    zeros = jnp.zeros((16,), jnp.float32)

    def zloop(i, _):
        hs[pl.ds(i * 16, 16)] = zeros
        hd[pl.ds(i * 16, 16)] = zeros
        return 0

    lax.fori_loop(0, NPAD // 16, zloop, 0)

    pltpu.sync_copy(src_hbm.at[pl.ds(wid * EW, EW)], sbuf)
    pltpu.sync_copy(dst_hbm.at[pl.ds(wid * EW, EW)], dbuf)

    ones = jnp.ones((16,), jnp.float32)

    def acc(i, _):
        sv = sbuf[pl.ds(i * 16, 16)]
        dv = dbuf[pl.ds(i * 16, 16)]
        plsc.addupdate_scatter(hs, [sv], ones)
        plsc.addupdate_scatter(hd, [dv], ones)
        return 0

    lax.fori_loop(0, EW // 16, acc, 0)

    pltpu.sync_copy(hs, shared.at[sid, 0])
    pltpu.sync_copy(hd, shared.at[sid, 1])
    plsc.subcore_barrier()

    base = sid * SL
    pltpu.sync_copy(shared.at[:, :, pl.ds(base, SL)], rbuf)

    def red(v, _):
        sacc = jnp.zeros((16,), jnp.float32)
        dacc = jnp.zeros((16,), jnp.float32)
        for r in range(NS):
            sacc = sacc + rbuf[r, 0, pl.ds(v * 16, 16)]
            dacc = dacc + rbuf[r, 1, pl.ds(v * 16, 16)]
        od[pl.ds(v * 16, 16)] = sacc
        oi[pl.ds(v * 16, 16)] = dacc
        return 0

    lax.fori_loop(0, SL // 16, red, 0)

    pltpu.sync_copy(od, out_hbm.at[2 * cid, pl.ds(base, SL)])
    pltpu.sync_copy(oi, out_hbm.at[2 * cid + 1, pl.ds(base, SL)])

    def zrow(i, _):
        od[pl.ds(i * 16, 16)] = zeros
        return 0

    lax.fori_loop(0, SL // 16, zrow, 0)
    pltpu.sync_copy(od, out_hbm.at[4 + 2 * cid, pl.ds(base, SL)])
    pltpu.sync_copy(od, out_hbm.at[5 + 2 * cid, pl.ds(base, SL)])


_hist_call = pl.kernel(
    _hist_body,
    out_type=jax.ShapeDtypeStruct((8, NPAD), jnp.float32),
    mesh=_mesh,
    compiler_params=pltpu.CompilerParams(needs_layout_passes=False),
    scratch_types=[
        pltpu.VMEM((EW,), jnp.int32),
        pltpu.VMEM((EW,), jnp.int32),
        pltpu.VMEM((NPAD,), jnp.float32),
        pltpu.VMEM((NPAD,), jnp.float32),
        pltpu.VMEM((NS, 2, SL), jnp.float32),
        pltpu.VMEM((SL,), jnp.float32),
        pltpu.VMEM((SL,), jnp.float32),
        pltpu.VMEM_SHARED((NS, 2, NPAD), jnp.float32),
    ],
)


def _scale_kernel(x_ref, w2_ref, hist_ref, h_ref):
    xb = x_ref[...]
    w2 = w2_ref[...]
    h = lax.dot_general(w2, xb, (((1,), (1,)), ((), ())),
                        preferred_element_type=jnp.float32)
    hist = hist_ref[...]
    deg = hist[0:1, :] + hist[2:3, :]
    a = lax.rsqrt(deg + 1.0)
    r = jnp.sqrt(jnp.sum(h * h, axis=0, keepdims=True))
    s = a / jnp.maximum(a * r, 1e-12)
    h_ref[...] = h * s


def _scale_call(x_pad, w2, hist):
    return pl.pallas_call(
        _scale_kernel,
        grid=(NBLK,),
        in_specs=[
            pl.BlockSpec((NB, D), lambda i: (i, 0)),
            pl.BlockSpec((D, D), lambda i: (0, 0)),
            pl.BlockSpec((8, NB), lambda i: (0, i)),
        ],
        out_specs=pl.BlockSpec((D, NB), lambda i: (0, i)),
        out_shape=jax.ShapeDtypeStruct((D, NPAD), jnp.float32),
    )(x_pad, w2, hist)


def _gmax_body(h_hbm, src_hbm, dst_hbm, agg_hbm, tflat, aflat, sbuf, dbuf,
               ob0, ob1, sem0, sem1):
    cid = lax.axis_index("c")
    sid = lax.axis_index("s")
    fb = sid * P
    ebase = cid * ESH
    NW2 = P // 2

    for w in range(NW2):
        pltpu.sync_copy(h_hbm.at[fb + 2 * w], ob0)
        pltpu.sync_copy(h_hbm.at[fb + 2 * w + 1], ob1)

        def pk(i, _):
            for u in range(8):
                va = ob0[pl.ds(i * 128 + u * 16, 16)]
                vb = ob1[pl.ds(i * 128 + u * 16, 16)]
                wd = plsc.bitcast(
                    plsc.pack(va, vb, format=plsc.PackFormat.INTERLEAVED),
                    jnp.int32)
                tflat[pl.ds(w * NPAD + i * 128 + u * 16, 16)] = wd
            return 0

        lax.fori_loop(0, NPAD // 128, pk, 0)

    negs = jnp.full((16,), NEG, jnp.float32)
    wneg = plsc.bitcast(
        plsc.pack(negs, negs, format=plsc.PackFormat.INTERLEAVED), jnp.int32)

    def zloop(i, _):
        for u in range(8):
            aflat[pl.ds(i * 128 + u * 16, 16)] = wneg
        return 0

    lax.fori_loop(0, NW2 * NPAD // 128, zloop, 0)

    sems = (sem0, sem1)

    def start(c, slot):
        pltpu.make_async_copy(
            src_hbm.at[pl.ds(ebase + c * CH2, CH2)],
            sbuf.at[pl.ds(slot * CH2, CH2)], sems[slot]).start()
        pltpu.make_async_copy(
            dst_hbm.at[pl.ds(ebase + c * CH2, CH2)],
            dbuf.at[pl.ds(slot * CH2, CH2)], sems[slot]).start()

    def wait(c, slot):
        pltpu.make_async_copy(
            src_hbm.at[pl.ds(ebase + c * CH2, CH2)],
            sbuf.at[pl.ds(slot * CH2, CH2)], sems[slot]).wait()
        pltpu.make_async_copy(
            dst_hbm.at[pl.ds(ebase + c * CH2, CH2)],
            dbuf.at[pl.ds(slot * CH2, CH2)], sems[slot]).wait()

    def unpack_word(wv):
        return plsc.unpack(plsc.bitcast(wv, jnp.bfloat16),
                           format=plsc.PackFormat.INTERLEAVED)

    def pack_word(a, b):
        return plsc.bitcast(
            plsc.pack(a, b, format=plsc.PackFormat.INTERLEAVED), jnp.int32)

    def do_chunk(slot):
        def grp(gi, _):
            svs, dvs = [], []
            dupm = None
            for j in range(GRP):
                k = slot * (CH2 // 16) + gi * GRP + j
                dvs.append(dbuf[pl.ds(k * 16, 16)])
                svs.append(sbuf[pl.ds(k * 16, 16)])
            for j in range(GRP):
                _, lastocc = plsc.scan_count(dvs[j])
                d = jnp.logical_not(lastocc)
                dupm = d if dupm is None else (dupm | d)

            for j in range(GRP):
                sv, dv = svs[j], dvs[j]

                @plsc.parallel_loop(0, NW2, 1, unroll=NW2)
                def _floop(w):
                    off = w * NPAD
                    gw = plsc.load_gather(tflat, [sv + off])
                    di = dv + off
                    aw = plsc.load_gather(aflat, [di])
                    m = jnp.maximum(plsc.bitcast(aw, jnp.bfloat16),
                                    plsc.bitcast(gw, jnp.bfloat16))
                    plsc.store_scatter(aflat, [di],
                                       plsc.bitcast(m, jnp.int32))

            @pl.when(jnp.any(dupm))
            def _fixup():
                for j in range(GRP):
                    k = slot * (CH2 // 16) + gi * GRP + j
                    dv = dbuf[pl.ds(k * 16, 16)]
                    sv = sbuf[pl.ds(k * 16, 16)]
                    gs = [unpack_word(plsc.load_gather(tflat, [sv + w * NPAD]))
                          for w in range(NW2)]
                    dis = [dv + w * NPAD for w in range(NW2)]
                    full = dv >= 0

                    def cond(ps):
                        m = ps[0]
                        for q in ps[1:]:
                            m = m | q
                        return jnp.any(m)

                    def body(ps):
                        out = []
                        for w in range(NW2):
                            g0, g1 = gs[w]
                            a0, a1 = unpack_word(
                                plsc.load_gather(aflat, [dis[w]]))
                            m = pack_word(jnp.maximum(a0, g0),
                                          jnp.maximum(a1, g1))
                            plsc.store_scatter(aflat, [dis[w]], m,
                                               mask=ps[2 * w] | ps[2 * w + 1])
                            b0, b1 = unpack_word(
                                plsc.load_gather(aflat, [dis[w]]))
                            out.append(b0 < g0)
                            out.append(b1 < g1)
                        return tuple(out)

                    lax.while_loop(cond, body,
                                   tuple(full for _ in range(P)))

            return 0

        lax.fori_loop(0, CH2 // 16 // GRP, grp, 0)

    start(0, 0)

    def pair(p, _):
        c0 = 2 * p
        start(c0 + 1, 1)
        wait(c0, 0)
        do_chunk(0)

        @pl.when(p + 1 < NCH2 // 2)
        def _pre():
            start(c0 + 2, 0)

        wait(c0 + 1, 1)
        do_chunk(1)
        return 0

    lax.fori_loop(0, NCH2 // 2, pair, 0)

    for w in range(NW2):
        def up(i, _):
            for u in range(8):
                a0, a1 = unpack_word(
                    aflat[pl.ds(w * NPAD + i * 128 + u * 16, 16)])
                ob0[pl.ds(i * 128 + u * 16, 16)] = a0
                ob1[pl.ds(i * 128 + u * 16, 16)] = a1
            return 0

        lax.fori_loop(0, NPAD // 128, up, 0)
        pltpu.sync_copy(ob0, agg_hbm.at[cid, fb + 2 * w])
        pltpu.sync_copy(ob1, agg_hbm.at[cid, fb + 2 * w + 1])


_gmax_call = pl.kernel(
    _gmax_body,
    out_type=jax.ShapeDtypeStruct((NC, D, NPAD), jnp.float32),
    mesh=_mesh,
    compiler_params=pltpu.CompilerParams(needs_layout_passes=False),
    scratch_types=(
        [pltpu.VMEM((P // 2 * NPAD,), jnp.int32),
         pltpu.VMEM((P // 2 * NPAD,), jnp.int32),
         pltpu.VMEM((2 * CH2,), jnp.int32),
         pltpu.VMEM((2 * CH2,), jnp.int32),
         pltpu.VMEM((NPAD,), jnp.float32),
         pltpu.VMEM((NPAD,), jnp.float32),
         pltpu.SemaphoreType.DMA,
         pltpu.SemaphoreType.DMA]
    ),
)


def _out_kernel(x_ref, w1_ref, hist_ref, agg_ref, o_ref):
    xb = x_ref[...]
    w1 = w1_ref[...]
    y = lax.dot_general(xb, w1, (((1,), (1,)), ((), ())),
                        preferred_element_type=jnp.float32)
    hist = hist_ref[...]
    inc = hist[1:2, :] + hist[3:4, :]
    agm = jnp.maximum(agg_ref[0], agg_ref[1])
    ag = jnp.where(inc > 0.0, agm, 0.0)
    ii = lax.broadcasted_iota(jnp.int32, (D, D), 0)
    jj = lax.broadcasted_iota(jnp.int32, (D, D), 1)
    eye = jnp.where(ii == jj, 1.0, 0.0)
    agt = lax.dot_general(ag, eye, (((0,), (0,)), ((), ())),
                          preferred_element_type=jnp.float32,
                          precision=lax.Precision.HIGHEST)
    o_ref[...] = ALPHA * y + agt


def _out_call(x_pad, w1, hist, agg):
    return pl.pallas_call(
        _out_kernel,
        grid=(NBLK,),
        in_specs=[
            pl.BlockSpec((NB, D), lambda i: (i, 0)),
            pl.BlockSpec((D, D), lambda i: (0, 0)),
            pl.BlockSpec((8, NB), lambda i: (0, i)),
            pl.BlockSpec((NC, D, NB), lambda i: (0, 0, i)),
        ],
        out_specs=pl.BlockSpec((NB, D), lambda i: (i, 0)),
        out_shape=jax.ShapeDtypeStruct((NPAD, D), jnp.float32),
    )(x_pad, w1, hist, agg)


@jax.jit
def kernel(x, edge_index, W1, W2):
    ei = jnp.asarray(edge_index, jnp.int32)
    src = ei[0]
    dst = ei[1]
    x_pad = jnp.pad(x, ((0, NPAD - N_NODES), (0, 0)))
    hist = _hist_call(src, dst)
    h_scaled = _scale_call(x_pad, W2, hist)
    agg = _gmax_call(h_scaled, src, dst)
    out = _out_call(x_pad, W1, hist, agg)
    return out[:N_NODES]


# speedup vs baseline: 1.1526x; 1.1526x over previous
import functools
import jax
import jax.numpy as jnp
from jax import lax
from jax.experimental import pallas as pl
from jax.experimental.pallas import tpu as pltpu
from jax.experimental.pallas import tpu_sc as plsc

N_NODES = 10000
N_EDGES = 320000
D = 128
ALPHA = 1.0

NPAD = 10240
NB = 1024
NBLK = NPAD // NB
NC, NS = 2, 16
NW = NC * NS
EW = N_EDGES // NW
SL = NPAD // NS
FPT = D // NW
CH = 3200
NCH = N_EDGES // CH
GRP = 5
P = 8
ESH = N_EDGES // NC
CH2 = 4000
NCH2 = ESH // CH2
NEG = -3.0e38

_mesh = plsc.VectorSubcoreMesh(core_axis_name="c", subcore_axis_name="s")


def _hist_body(src_hbm, dst_hbm, out_hbm, sbuf, dbuf, hs, hd, rbuf, od, oi, shared):
    cid = lax.axis_index("c")
    sid = lax.axis_index("s")
    wid = cid * NS + sid

    zeros = jnp.zeros((16,), jnp.float32)

    def zloop(i, _):
        hs[pl.ds(i * 16, 16)] = zeros
        hd[pl.ds(i * 16, 16)] = zeros
        return 0

    lax.fori_loop(0, NPAD // 16, zloop, 0)

    pltpu.sync_copy(src_hbm.at[pl.ds(wid * EW, EW)], sbuf)
    pltpu.sync_copy(dst_hbm.at[pl.ds(wid * EW, EW)], dbuf)

    ones = jnp.ones((16,), jnp.float32)

    def acc(i, _):
        sv = sbuf[pl.ds(i * 16, 16)]
        dv = dbuf[pl.ds(i * 16, 16)]
        plsc.addupdate_scatter(hs, [sv], ones)
        plsc.addupdate_scatter(hd, [dv], ones)
        return 0

    lax.fori_loop(0, EW // 16, acc, 0)

    pltpu.sync_copy(hs, shared.at[sid, 0])
    pltpu.sync_copy(hd, shared.at[sid, 1])
    plsc.subcore_barrier()

    base = sid * SL
    pltpu.sync_copy(shared.at[:, :, pl.ds(base, SL)], rbuf)

    def red(v, _):
        sacc = jnp.zeros((16,), jnp.float32)
        dacc = jnp.zeros((16,), jnp.float32)
        for r in range(NS):
            sacc = sacc + rbuf[r, 0, pl.ds(v * 16, 16)]
            dacc = dacc + rbuf[r, 1, pl.ds(v * 16, 16)]
        od[pl.ds(v * 16, 16)] = sacc
        oi[pl.ds(v * 16, 16)] = dacc
        return 0

    lax.fori_loop(0, SL // 16, red, 0)

    pltpu.sync_copy(od, out_hbm.at[2 * cid, pl.ds(base, SL)])
    pltpu.sync_copy(oi, out_hbm.at[2 * cid + 1, pl.ds(base, SL)])

    def zrow(i, _):
        od[pl.ds(i * 16, 16)] = zeros
        return 0

    lax.fori_loop(0, SL // 16, zrow, 0)
    pltpu.sync_copy(od, out_hbm.at[4 + 2 * cid, pl.ds(base, SL)])
    pltpu.sync_copy(od, out_hbm.at[5 + 2 * cid, pl.ds(base, SL)])


_hist_call = pl.kernel(
    _hist_body,
    out_type=jax.ShapeDtypeStruct((8, NPAD), jnp.float32),
    mesh=_mesh,
    compiler_params=pltpu.CompilerParams(needs_layout_passes=False),
    scratch_types=[
        pltpu.VMEM((EW,), jnp.int32),
        pltpu.VMEM((EW,), jnp.int32),
        pltpu.VMEM((NPAD,), jnp.float32),
        pltpu.VMEM((NPAD,), jnp.float32),
        pltpu.VMEM((NS, 2, SL), jnp.float32),
        pltpu.VMEM((SL,), jnp.float32),
        pltpu.VMEM((SL,), jnp.float32),
        pltpu.VMEM_SHARED((NS, 2, NPAD), jnp.float32),
    ],
)


def _scale_kernel(x_ref, w2_ref, hist_ref, h_ref):
    xb = x_ref[...]
    w2 = w2_ref[...]
    h = lax.dot_general(w2, xb, (((1,), (1,)), ((), ())),
                        preferred_element_type=jnp.float32)
    hist = hist_ref[...]
    deg = hist[0:1, :] + hist[2:3, :]
    a = lax.rsqrt(deg + 1.0)
    r = jnp.sqrt(jnp.sum(h * h, axis=0, keepdims=True))
    s = a / jnp.maximum(a * r, 1e-12)
    h_ref[...] = h * s


def _scale_call(x_pad, w2, hist):
    return pl.pallas_call(
        _scale_kernel,
        grid=(NBLK,),
        in_specs=[
            pl.BlockSpec((NB, D), lambda i: (i, 0)),
            pl.BlockSpec((D, D), lambda i: (0, 0)),
            pl.BlockSpec((8, NB), lambda i: (0, i)),
        ],
        out_specs=pl.BlockSpec((D, NB), lambda i: (0, i)),
        out_shape=jax.ShapeDtypeStruct((D, NPAD), jnp.float32),
    )(x_pad, w2, hist)


def _gmax_body(h_hbm, src_hbm, dst_hbm, agg_hbm, tflat, aflat, sbuf, dbuf,
               ob0, ob1, sem0, sem1):
    cid = lax.axis_index("c")
    sid = lax.axis_index("s")
    fb = sid * P
    ebase = cid * ESH
    NW2 = P // 2

    for w in range(NW2):
        pltpu.sync_copy(h_hbm.at[fb + 2 * w], ob0)
        pltpu.sync_copy(h_hbm.at[fb + 2 * w + 1], ob1)

        def pk(i, _):
            for u in range(8):
                va = ob0[pl.ds(i * 128 + u * 16, 16)]
                vb = ob1[pl.ds(i * 128 + u * 16, 16)]
                wd = plsc.bitcast(
                    plsc.pack(va, vb, format=plsc.PackFormat.INTERLEAVED),
                    jnp.int32)
                tflat[pl.ds(w * NPAD + i * 128 + u * 16, 16)] = wd
            return 0

        lax.fori_loop(0, NPAD // 128, pk, 0)

    negs = jnp.full((16,), NEG, jnp.float32)
    wneg = plsc.bitcast(
        plsc.pack(negs, negs, format=plsc.PackFormat.INTERLEAVED), jnp.int32)

    def zloop(i, _):
        for u in range(8):
            aflat[pl.ds(i * 128 + u * 16, 16)] = wneg
        return 0

    lax.fori_loop(0, NW2 * NPAD // 128, zloop, 0)

    sems = (sem0, sem1)

    def start(c, slot):
        pltpu.make_async_copy(
            src_hbm.at[pl.ds(ebase + c * CH2, CH2)],
            sbuf.at[pl.ds(slot * CH2, CH2)], sems[slot]).start()
        pltpu.make_async_copy(
            dst_hbm.at[pl.ds(ebase + c * CH2, CH2)],
            dbuf.at[pl.ds(slot * CH2, CH2)], sems[slot]).start()

    def wait(c, slot):
        pltpu.make_async_copy(
            src_hbm.at[pl.ds(ebase + c * CH2, CH2)],
            sbuf.at[pl.ds(slot * CH2, CH2)], sems[slot]).wait()
        pltpu.make_async_copy(
            dst_hbm.at[pl.ds(ebase + c * CH2, CH2)],
            dbuf.at[pl.ds(slot * CH2, CH2)], sems[slot]).wait()

    def unpack_word(wv):
        return plsc.unpack(plsc.bitcast(wv, jnp.bfloat16),
                           format=plsc.PackFormat.INTERLEAVED)

    def pack_word(a, b):
        return plsc.bitcast(
            plsc.pack(a, b, format=plsc.PackFormat.INTERLEAVED), jnp.int32)

    def do_chunk(slot):
        def grp(gi, _):
            svs, dvs = [], []
            dupm = None
            for j in range(GRP):
                k = slot * (CH2 // 16) + gi * GRP + j
                dvs.append(dbuf[pl.ds(k * 16, 16)])
                svs.append(sbuf[pl.ds(k * 16, 16)])
            for j in range(GRP):
                _, lastocc = plsc.scan_count(dvs[j])
                d = jnp.logical_not(lastocc)
                dupm = d if dupm is None else (dupm | d)

            for j in range(GRP):
                sv, dv = svs[j], dvs[j]

                @plsc.parallel_loop(0, NW2, 1, unroll=NW2)
                def _floop(w):
                    off = w * NPAD
                    gw = plsc.load_gather(tflat, [sv + off])
                    di = dv + off
                    aw = plsc.load_gather(aflat, [di])
                    m = jnp.maximum(plsc.bitcast(aw, jnp.bfloat16),
                                    plsc.bitcast(gw, jnp.bfloat16))
                    plsc.store_scatter(aflat, [di],
                                       plsc.bitcast(m, jnp.int32))

            @pl.when(jnp.any(dupm))
            def _fixup():
                for j in range(GRP):
                    k = slot * (CH2 // 16) + gi * GRP + j
                    dv = dbuf[pl.ds(k * 16, 16)]
                    sv = sbuf[pl.ds(k * 16, 16)]
                    gs = [unpack_word(plsc.load_gather(tflat, [sv + w * NPAD]))
                          for w in range(NW2)]
                    dis = [dv + w * NPAD for w in range(NW2)]
                    full = dv >= 0

                    def cond(ps):
                        m = ps[0]
                        for q in ps[1:]:
                            m = m | q
                        return jnp.any(m)

                    def body(ps):
                        out = []
                        for w in range(NW2):
                            g0, g1 = gs[w]
                            a0, a1 = unpack_word(
                                plsc.load_gather(aflat, [dis[w]]))
                            m = pack_word(jnp.maximum(a0, g0),
                                          jnp.maximum(a1, g1))
                            plsc.store_scatter(aflat, [dis[w]], m,
                                               mask=ps[2 * w] | ps[2 * w + 1])
                            b0, b1 = unpack_word(
                                plsc.load_gather(aflat, [dis[w]]))
                            out.append(b0 < g0)
                            out.append(b1 < g1)
                        return tuple(out)

                    lax.while_loop(cond, body,
                                   tuple(full for _ in range(P)))

            return 0

        lax.fori_loop(0, CH2 // 16 // GRP, grp, 0)

    start(0, 0)

    def pair(p, _):
        c0 = 2 * p
        start(c0 + 1, 1)
        wait(c0, 0)
        do_chunk(0)

        @pl.when(p + 1 < NCH2 // 2)
        def _pre():
            start(c0 + 2, 0)

        wait(c0 + 1, 1)
        do_chunk(1)
        return 0

    lax.fori_loop(0, NCH2 // 2, pair, 0)

    for w in range(NW2):
        def up(i, _):
            for u in range(8):
                a0, a1 = unpack_word(
                    aflat[pl.ds(w * NPAD + i * 128 + u * 16, 16)])
                ob0[pl.ds(i * 128 + u * 16, 16)] = a0
                ob1[pl.ds(i * 128 + u * 16, 16)] = a1
            return 0

        lax.fori_loop(0, NPAD // 128, up, 0)
        pltpu.sync_copy(ob0, agg_hbm.at[cid, fb + 2 * w])
        pltpu.sync_copy(ob1, agg_hbm.at[cid, fb + 2 * w + 1])


_gmax_call = pl.kernel(
    _gmax_body,
    out_type=jax.ShapeDtypeStruct((NC, D, NPAD), jnp.float32),
    mesh=_mesh,
    compiler_params=pltpu.CompilerParams(needs_layout_passes=False),
    scratch_types=(
        [pltpu.VMEM((P // 2 * NPAD,), jnp.int32),
         pltpu.VMEM((P // 2 * NPAD,), jnp.int32),
         pltpu.VMEM((2 * CH2,), jnp.int32),
         pltpu.VMEM((2 * CH2,), jnp.int32),
         pltpu.VMEM((NPAD,), jnp.float32),
         pltpu.VMEM((NPAD,), jnp.float32),
         pltpu.SemaphoreType.DMA,
         pltpu.SemaphoreType.DMA]
    ),
)


def _out_kernel(x_ref, w1_ref, hist_ref, agg_ref, o_ref):
    xb = x_ref[...]
    w1 = w1_ref[...]
    y = lax.dot_general(xb, w1, (((1,), (1,)), ((), ())),
                        preferred_element_type=jnp.float32)
    hist = hist_ref[...]
    inc = hist[1:2, :] + hist[3:4, :]
    agm = jnp.maximum(agg_ref[0], agg_ref[1])
    ag = jnp.where(inc > 0.0, agm, 0.0)
    ii = lax.broadcasted_iota(jnp.int32, (D, D), 0)
    jj = lax.broadcasted_iota(jnp.int32, (D, D), 1)
    eye = jnp.where(ii == jj, 1.0, 0.0)
    agt = lax.dot_general(ag, eye, (((0,), (0,)), ((), ())),
                          preferred_element_type=jnp.float32,
                          precision=lax.Precision.HIGHEST)
    o_ref[...] = ALPHA * y + agt


def _out_call(x_pad, w1, hist, agg):
    return pl.pallas_call(
        _out_kernel,
        grid=(NBLK,),
        in_specs=[
            pl.BlockSpec((NB, D), lambda i: (i, 0)),
            pl.BlockSpec((D, D), lambda i: (0, 0)),
            pl.BlockSpec((8, NB), lambda i: (0, i)),
            pl.BlockSpec((NC, D, NB), lambda i: (0, 0, i)),
        ],
        out_specs=pl.BlockSpec((NB, D), lambda i: (i, 0)),
        out_shape=jax.ShapeDtypeStruct((NPAD, D), jnp.float32),
    )(x_pad, w1, hist, agg)


@jax.jit
def kernel(x, edge_index, W1, W2):
    ei = jnp.asarray(edge_index, jnp.int32)
    src = ei[0]
    dst = ei[1]
    x_pad = jnp.pad(x, ((0, NPAD - N_NODES), (0, 0)))
    hist = _hist_call(src, dst)
    h_scaled = _scale_call(x_pad, W2, hist)
    agg = _gmax_call(h_scaled, src, dst)
    out = _out_call(x_pad, W1, hist, agg)
    return out[:N_NODES]
